# baseline (speedup 1.0000x reference)
import functools

import jax
import jax.numpy as jnp
from jax.experimental import pallas as pl
from jax.experimental.pallas import tpu as pltpu

_BUFFER_SIZE = 100000
_D = 784
_MAXVAL = 256
_NUM_CHAINS = 1024
_N_NEW = 51
_N_OLD = _NUM_CHAINS - _N_NEW
_GRP = 128


def _sweep_body(idx_s, buf_ref, new_ref, out_ref):
    k = pl.program_id(0)
    kl = k % _GRP
    lane = jax.lax.broadcasted_iota(jnp.int32, (_D, _GRP), 1)

    @pl.when(k < _N_NEW)
    def _():
        out_ref[...] = jnp.where(lane == kl, new_ref[...], out_ref[...])

    @pl.when(k >= _N_NEW)
    def _():
        l = idx_s[k] % _GRP
        rot = pltpu.roll(buf_ref[...], (kl - l) % _GRP, 1)
        out_ref[...] = jnp.where(lane == kl, rot, out_ref[...])


@functools.partial(jax.jit, static_argnames=())
def _assemble(buf_t, new_t, idx_full):
    grid_spec = pltpu.PrefetchScalarGridSpec(
        num_scalar_prefetch=1,
        grid=(_NUM_CHAINS,),
        in_specs=[
            pl.BlockSpec((_D, _GRP), lambda k, s: (0, s[k] // _GRP)),
            pl.BlockSpec((_D, _GRP), lambda k, s: (0, 0)),
        ],
        out_specs=pl.BlockSpec((_D, _GRP), lambda k, s: (0, k // _GRP)),
    )
    out_t = pl.pallas_call(
        _sweep_body,
        grid_spec=grid_spec,
        out_shape=jax.ShapeDtypeStruct((_D, _NUM_CHAINS), jnp.int32),
        compiler_params=pltpu.CompilerParams(
            dimension_semantics=("arbitrary",),
        ),
    )(idx_full, buf_t, new_t)
    return out_t.T


def kernel(buffer, key):
    key, subkey = jax.random.split(key, 2)
    new_samples = jax.random.randint(
        subkey, minval=0, maxval=_MAXVAL, shape=(_N_NEW, _D)
    )
    key, subkey = jax.random.split(key, 2)
    idx = jax.random.choice(subkey, _BUFFER_SIZE, shape=(_N_OLD,))
    idx_full = jnp.concatenate(
        [jnp.zeros((_N_NEW,), jnp.int32), idx.astype(jnp.int32)]
    )
    new_t = jnp.pad(new_samples.T, ((0, 0), (0, _GRP - _N_NEW)))
    return _assemble(buffer.T, new_t, idx_full)
